# baseline (speedup 1.0000x reference)
import functools

import jax
import jax.numpy as jnp
from jax import lax
from jax.experimental import pallas as pl
from jax.experimental.pallas import tpu as pltpu
from jax.experimental.pallas import tpu_sc as plsc


def _tc_body(x_ref, pe_ref, o_ref):
    o_ref[...] = x_ref[...] + pe_ref[...]


def _tc_add(x, pos_emb, block_s=2048):
    B, S, D = x.shape
    grid = (S // block_s, B)
    return pl.pallas_call(
        _tc_body,
        grid=grid,
        in_specs=[
            pl.BlockSpec((1, block_s, D), lambda s, b: (b, s, 0)),
            pl.BlockSpec((block_s, D), lambda s, b: (s, 0)),
        ],
        out_specs=pl.BlockSpec((1, block_s, D), lambda s, b: (b, s, 0)),
        out_shape=jax.ShapeDtypeStruct(x.shape, x.dtype),
    )(x, pos_emb)


_LANES = 16
_NW = 32


def _sc_add(x2d, pos_emb, chunk_rows=32):
    R, D = x2d.shape
    S = pos_emb.shape[0]
    rows_per_w = R // _NW
    n_chunks = rows_per_w // chunk_rows
    mesh = plsc.VectorSubcoreMesh(core_axis_name="c", subcore_axis_name="s")

    def body(x_hbm, pe_hbm, out_hbm, xbuf, pebuf):
        wid = lax.axis_index("s") * 2 + lax.axis_index("c")
        base = wid * rows_per_w

        def chunk(ci, carry):
            row0 = base + ci * chunk_rows
            s0 = lax.rem(row0, S)
            pltpu.sync_copy(x_hbm.at[pl.ds(row0, chunk_rows)], xbuf)
            pltpu.sync_copy(pe_hbm.at[pl.ds(s0, chunk_rows)], pebuf)

            def row_add(r, carry2):
                for k in range(D // _LANES):
                    sl = pl.ds(k * _LANES, _LANES)
                    xbuf[r, sl] = xbuf[r, sl] + pebuf[r, sl]
                return carry2

            lax.fori_loop(0, chunk_rows, row_add, 0, unroll=False)
            pltpu.sync_copy(xbuf, out_hbm.at[pl.ds(row0, chunk_rows)])
            return carry

        lax.fori_loop(0, n_chunks, chunk, 0, unroll=False)

    fn = pl.kernel(
        body,
        out_type=jax.ShapeDtypeStruct((R, D), x2d.dtype),
        mesh=mesh,
        scratch_types=[
            pltpu.VMEM((chunk_rows, D), jnp.float32),
            pltpu.VMEM((chunk_rows, D), jnp.float32),
        ],
    )
    return fn(x2d, pos_emb)


def kernel(x, pos_emb):
    B, S, D = x.shape
    x2d = x.reshape(B * S, D)
    out2d = _sc_add(x2d, pos_emb[:S])
    return out2d.reshape(B, S, D)


# speedup vs baseline: 1.8308x; 1.8308x over previous
import jax
import jax.numpy as jnp
from jax import lax
from jax.experimental import pallas as pl
from jax.experimental.pallas import tpu as pltpu
from jax.experimental.pallas import tpu_sc as plsc


_LANES = 16
_NW = 32


def _sc_add(x2d, pos_emb, chunk_rows=16):
    R, D = x2d.shape
    S = pos_emb.shape[0]
    B = R // S
    C = chunk_rows
    s_per_w = S // _NW
    n_chunks = s_per_w // C
    mesh = plsc.VectorSubcoreMesh(core_axis_name="c", subcore_axis_name="s")

    def body(x_hbm, pe_hbm, out_hbm, xbuf, pebuf, load_sems, store_sems):
        wid = lax.axis_index("s") * 2 + lax.axis_index("c")
        s_base = wid * s_per_w

        def load_descs(i):
            slot = i % 2
            s0 = s_base + i * C
            descs = [
                pltpu.make_async_copy(
                    pe_hbm.at[pl.ds(s0, C)], pebuf.at[slot], load_sems.at[slot]
                )
            ]
            for b in range(B):
                descs.append(
                    pltpu.make_async_copy(
                        x_hbm.at[pl.ds(b * S + s0, C)],
                        xbuf.at[slot, b],
                        load_sems.at[slot],
                    )
                )
            return descs

        def store_descs(i):
            slot = i % 2
            s0 = s_base + i * C
            return [
                pltpu.make_async_copy(
                    xbuf.at[slot, b],
                    out_hbm.at[pl.ds(b * S + s0, C)],
                    store_sems.at[slot],
                )
                for b in range(B)
            ]

        def compute(i):
            slot = i % 2

            def row_add(r, carry):
                for k in range(D // _LANES):
                    sl = pl.ds(k * _LANES, _LANES)
                    pv = pebuf[slot, r, sl]
                    for b in range(B):
                        plsc.addupdate(xbuf.at[slot, b, r, sl], pv)
                return carry

            lax.fori_loop(0, C, row_add, 0, unroll=False)

        for d in load_descs(0):
            d.start()
        for i in range(n_chunks):
            if i + 1 < n_chunks:
                if i >= 1:
                    for d in store_descs(i - 1):
                        d.wait()
                for d in load_descs(i + 1):
                    d.start()
            for d in load_descs(i):
                d.wait()
            compute(i)
            for d in store_descs(i):
                d.start()
        for i in (n_chunks - 2, n_chunks - 1):
            for d in store_descs(i):
                d.wait()

    fn = pl.kernel(
        body,
        out_type=jax.ShapeDtypeStruct((R, D), x2d.dtype),
        mesh=mesh,
        scratch_types=[
            pltpu.VMEM((2, B, C, D), jnp.float32),
            pltpu.VMEM((2, C, D), jnp.float32),
            pltpu.SemaphoreType.DMA((2,)),
            pltpu.SemaphoreType.DMA((2,)),
        ],
    )
    return fn(x2d, pos_emb)


def kernel(x, pos_emb):
    B, S, D = x.shape
    x2d = x.reshape(B * S, D)
    out2d = _sc_add(x2d, pos_emb[:S])
    return out2d.reshape(B, S, D)


# speedup vs baseline: 2.1366x; 1.1670x over previous
import jax
import jax.numpy as jnp
from jax import lax
from jax.experimental import pallas as pl
from jax.experimental.pallas import tpu as pltpu
from jax.experimental.pallas import tpu_sc as plsc


_LANES = 16
_NW = 32
_RING = 4


def _sc_add(x2d, pos_emb, chunk_rows=8):
    R, D = x2d.shape
    S = pos_emb.shape[0]
    B = R // S
    C = chunk_rows
    s_per_w = S // _NW
    n_chunks = s_per_w // C
    n_groups = n_chunks // _RING
    mesh = plsc.VectorSubcoreMesh(core_axis_name="c", subcore_axis_name="s")

    def body(x_hbm, pe_hbm, out_hbm, xbuf, pebuf, load_sems, store_sems):
        wid = lax.axis_index("s") * 2 + lax.axis_index("c")
        s_base = wid * s_per_w

        def load_descs(k, slot):
            s0 = s_base + k * C
            descs = [
                pltpu.make_async_copy(
                    pe_hbm.at[pl.ds(s0, C)], pebuf.at[slot], load_sems.at[slot]
                )
            ]
            for b in range(B):
                descs.append(
                    pltpu.make_async_copy(
                        x_hbm.at[pl.ds(b * S + s0, C)],
                        xbuf.at[slot, b],
                        load_sems.at[slot],
                    )
                )
            return descs

        def store_descs(k, slot):
            s0 = s_base + k * C
            return [
                pltpu.make_async_copy(
                    xbuf.at[slot, b],
                    out_hbm.at[pl.ds(b * S + s0, C)],
                    store_sems.at[slot],
                )
                for b in range(B)
            ]

        def compute(slot):
            def row_add(r, carry):
                for k in range(D // _LANES):
                    sl = pl.ds(k * _LANES, _LANES)
                    pv = pebuf[slot, r, sl]
                    for b in range(B):
                        plsc.addupdate(xbuf.at[slot, b, r, sl], pv)
                return carry

            lax.fori_loop(0, C, row_add, 0, unroll=False)

        for d in load_descs(0, 0):
            d.start()
        for d in load_descs(1, 1):
            d.start()

        def group(g, carry):
            for j in range(_RING):
                k = g * _RING + j
                slot_ahead = (j + 2) % _RING

                @pl.when(k >= 2)
                def _():
                    for d in store_descs(k - 2, slot_ahead):
                        d.wait()

                @pl.when(k + 2 < n_chunks)
                def _():
                    for d in load_descs(k + 2, slot_ahead):
                        d.start()

                for d in load_descs(k, j):
                    d.wait()
                compute(j)
                for d in store_descs(k, j):
                    d.start()
            return carry

        lax.fori_loop(0, n_groups, group, 0, unroll=False)
        for k, slot in ((n_chunks - 2, _RING - 2), (n_chunks - 1, _RING - 1)):
            for d in store_descs(k, slot):
                d.wait()

    fn = pl.kernel(
        body,
        out_type=jax.ShapeDtypeStruct((R, D), x2d.dtype),
        mesh=mesh,
        scratch_types=[
            pltpu.VMEM((_RING, B, C, D), jnp.float32),
            pltpu.VMEM((_RING, C, D), jnp.float32),
            pltpu.SemaphoreType.DMA((_RING,)),
            pltpu.SemaphoreType.DMA((_RING,)),
        ],
    )
    return fn(x2d, pos_emb)


def kernel(x, pos_emb):
    B, S, D = x.shape
    x2d = x.reshape(B * S, D)
    out2d = _sc_add(x2d, pos_emb[:S])
    return out2d.reshape(B, S, D)
